# baseline (speedup 1.0000x reference)
import jax
import jax.numpy as jnp
from jax.experimental import pallas as pl
from jax.experimental.pallas import tpu as pltpu

_NPIECE = 4


def _body(x_ref, cb_ref, out_hbm, out_v, sem_o):
    M, D = x_ref.shape
    K = cb_ref.shape[0]
    H = M // _NPIECE
    cb = cb_ref[...]
    c2 = jnp.sum(cb * cb, axis=1)[None, :]
    cps = []
    for h in range(_NPIECE):
        rows = pl.ds(h * H, H)
        xv = x_ref[rows, :]
        x2 = jnp.sum(xv * xv, axis=1, keepdims=True)
        xc = jax.lax.dot_general(xv, cb, (((1,), (1,)), ((), ())),
                                 preferred_element_type=jnp.float32)
        dist = x2 + c2 - 2.0 * xc
        mins = jnp.min(dist, axis=1, keepdims=True)
        kio = (jax.lax.broadcasted_iota(jnp.int32, (H, K), 1)
               .astype(jnp.float32))
        idx = jnp.min(jnp.where(dist == mins, kio, jnp.float32(K)),
                      axis=1, keepdims=True)
        onehot = jnp.where(kio == idx, jnp.float32(1), jnp.float32(0))
        out_v[rows, :] = jax.lax.dot_general(
            onehot, cb, (((1,), (0,)), ((), ())),
            preferred_element_type=jnp.float32)
        cp = pltpu.make_async_copy(out_v.at[rows], out_hbm.at[rows],
                                   sem_o.at[h])
        cp.start()
        cps.append(cp)
    for cp in cps:
        cp.wait()


def kernel(x, codebook):
    N, C, H, W = x.shape
    K = codebook.shape[0]
    D = H * W
    M = N * C
    x_flat = x.reshape(M, D)
    cb_flat = codebook.reshape(K, D)
    out = pl.pallas_call(
        _body,
        out_specs=pl.BlockSpec(memory_space=pl.ANY),
        out_shape=jax.ShapeDtypeStruct((M, D), jnp.float32),
        scratch_shapes=[
            pltpu.VMEM((M, D), jnp.float32),
            pltpu.SemaphoreType.DMA((_NPIECE,)),
        ],
    )(x_flat, cb_flat)
    return out.reshape(N, C, H, W)
